# baseline (speedup 1.0000x reference)
import jax
import jax.numpy as jnp
from jax.experimental import pallas as pl


def _add_kernel(we_ref, pos_ref, out_ref):
    out_ref[...] = we_ref[...] + pos_ref[...][None, :, :]


def kernel(word_embeddings, pos_table):
    B, S, D = word_embeddings.shape
    BS = 1024
    grid = (S // BS, B // 2)
    return pl.pallas_call(
        _add_kernel,
        grid=grid,
        in_specs=[
            pl.BlockSpec((2, BS, D), lambda s, b: (b, s, 0)),
            pl.BlockSpec((BS, D), lambda s, b: (s, 0)),
        ],
        out_specs=pl.BlockSpec((2, BS, D), lambda s, b: (b, s, 0)),
        out_shape=jax.ShapeDtypeStruct((B, S, D), word_embeddings.dtype),
    )(word_embeddings, pos_table)
